# Initial kernel scaffold; baseline (speedup 1.0000x reference)
import jax
import jax.numpy as jnp
from jax.experimental import pallas as pl


def kernel(vertices, cat_id, d0, w1, b1, d1, g1, be1, w2, b2, d2, g2, be2, w3, b3, d3, g3, be3, w4, b4, d4):
    raise NotImplementedError("write your pallas kernel here")


# speedup vs baseline: 7.9173x; 7.9173x over previous
import functools

import jax
import jax.numpy as jnp
from jax.experimental import pallas as pl
from jax.experimental.pallas import tpu as pltpu

S = 7
OBJ_C = 6
INF = float("inf")


def _topk_smallest(dist, k):
    v, w = dist.shape
    lane = jax.lax.broadcasted_iota(jnp.int32, (v, w), 1)
    picks = []
    d = dist
    for _ in range(k):
        rowmin = jnp.min(d, axis=1, keepdims=True)
        pick = jnp.min(jnp.where(d == rowmin, lane, w), axis=1, keepdims=True)
        picks.append(pick)
        d = jnp.where(lane == pick, INF, d)
    return picks


def _onehot(pick, w):
    v = pick.shape[0]
    lane = jax.lax.broadcasted_iota(jnp.int32, (v, w), 1)
    return (lane == pick).astype(jnp.float32)


def _gather_rows(values, pick):
    oh = _onehot(pick, values.shape[0])
    return jnp.dot(oh, values, preferred_element_type=jnp.float32)


def _split3(values):
    def top16(x):
        u = jax.lax.bitcast_convert_type(x, jnp.uint32)
        return jax.lax.bitcast_convert_type(u & jnp.uint32(0xFFFF0000),
                                            jnp.float32)
    hi = top16(values)
    rem = values - hi
    mid = top16(rem)
    lo = rem - mid
    return hi, mid, lo


def _gather_rows_exact(values, pick):
    oh = _onehot(pick, values.shape[0])
    hi, mid, lo = _split3(values)
    g = jnp.dot(oh, hi, preferred_element_type=jnp.float32)
    g = g + jnp.dot(oh, mid, preferred_element_type=jnp.float32)
    return g + jnp.dot(oh, lo, preferred_element_type=jnp.float32)


def _inner3(a, bt):
    return ((a[:, 0:1] * bt[0:1, :] + a[:, 1:2] * bt[1:2, :])
            + a[:, 2:3] * bt[2:3, :])


def _self_dist(v, vt):
    inner = jnp.dot(v, vt, preferred_element_type=jnp.float32)
    quad_row = jnp.sum(vt * vt, axis=0, keepdims=True)
    quad_col = jnp.sum(v * v, axis=1, keepdims=True)
    return (-2.0 * inner + quad_row) + quad_col


def _cross_dist(target, source_t):
    inner = jnp.dot(target, source_t, preferred_element_type=jnp.float32)
    s2 = jnp.sum(source_t * source_t, axis=0, keepdims=True)
    t2 = jnp.sum(target * target, axis=1, keepdims=True)
    return (s2 + t2) - 2.0 * inner


def _norm_dirs(d):
    n = jnp.sqrt(jnp.sum(d * d, axis=0, keepdims=True))
    return d / jnp.maximum(n, 1e-12)


def _neighbor_dir_norm(verts, pick, verts_split):
    oh = _onehot(pick, verts.shape[0])
    hi, mid, lo = verts_split
    nbr = jnp.dot(oh, hi, preferred_element_type=jnp.float32)
    nbr = nbr + jnp.dot(oh, mid, preferred_element_type=jnp.float32)
    nbr = nbr + jnp.dot(oh, lo, preferred_element_type=jnp.float32)
    direction = nbr - verts
    n = jnp.sqrt(jnp.sum(direction * direction, axis=1, keepdims=True))
    return direction / jnp.maximum(n, 1e-12)


def _group_sum(acc, out_c):
    total = acc[:, 0:out_c]
    for g in range(1, S):
        total = total + acc[:, g * out_c:(g + 1) * out_c]
    return total


def _s1_body(verts_ref, vt_ref, d0_ref, w1_ref, b1_ref, d1_ref,
             fm0_ref, y1_ref, nidx_ref):
    verts = verts_ref[0]
    dist = _self_dist(verts, vt_ref[0])
    picks = _topk_smallest(dist, 11)
    nbrs = picks[1:]
    nidx_ref[0] = jnp.concatenate(nbrs[:8], axis=1)

    sdn0 = _norm_dirs(d0_ref[...])
    sdn1 = _norm_dirs(d1_ref[...])
    vsplit = _split3(verts)
    ndns = [_neighbor_dir_norm(verts, p, vsplit) for p in nbrs]

    acc0 = jnp.full((verts.shape[0], S * 128), -INF, jnp.float32)
    for ndn in ndns:
        theta = jax.nn.relu(jnp.dot(ndn, sdn0,
                                    preferred_element_type=jnp.float32))
        acc0 = jnp.maximum(acc0, theta)
    fm0 = jax.nn.relu(_group_sum(acc0, 128))
    fm0_ref[0] = fm0

    fout = jnp.dot(fm0, w1_ref[...],
                   preferred_element_type=jnp.float32) + b1_ref[0][None, :]
    center = fout[:, :128]
    support = fout[:, 128:]
    acc1 = jnp.full((verts.shape[0], S * 128), -INF, jnp.float32)
    for p, ndn in zip(nbrs, ndns):
        theta = jax.nn.relu(jnp.dot(ndn, sdn1,
                                    preferred_element_type=jnp.float32))
        fs = _gather_rows(support, p)
        acc1 = jnp.maximum(acc1, theta * fs)
    y1_ref[0] = center + _group_sum(acc1, 128)


def _bn_body(y_ref, g_ref, be_ref, out_ref):
    y = y_ref[...]
    bs, v, c = y.shape
    y2 = y.reshape(bs * v, c)
    mu = jnp.mean(y2, axis=0, keepdims=True)
    var = jnp.mean((y2 - mu) ** 2, axis=0, keepdims=True)
    out = (y2 - mu) / jnp.sqrt(var + 1e-5) * g_ref[0][None, :] + be_ref[0][None, :]
    out_ref[...] = jax.nn.relu(out).reshape(bs, v, c)


def _pool_body(fm_ref, nidx_ref, out_ref):
    fm = fm_ref[0]
    acc = jnp.full((nidx_ref.shape[1], fm.shape[1]), -INF, jnp.float32)
    for k in range(4):
        pick = nidx_ref[0][:, k:k + 1]
        acc = jnp.maximum(acc, _gather_rows(fm, pick))
    out_ref[0] = acc


def _conv_body(verts_ref, vt_ref, feat_ref, w_ref, b_ref, d_ref, *refs,
               out_c, n_nbr, knn, with_global):
    i = 0
    if knn:
        y_ref = refs[i]; i += 1
        nidx_out_ref = refs[i]; i += 1
        nidx_in_ref = None
    else:
        nidx_in_ref = refs[i]; i += 1
        y_ref = refs[i]; i += 1
    g_ref = refs[i] if with_global else None

    verts = verts_ref[0]
    if knn:
        dist = _self_dist(verts, vt_ref[0])
        nbrs = _topk_smallest(dist, n_nbr + 1)[1:]
        if nidx_out_ref is not None:
            nidx_out_ref[0] = jnp.concatenate(nbrs, axis=1)
    else:
        nbrs = [nidx_in_ref[0][:, k:k + 1] for k in range(n_nbr)]

    sdn = _norm_dirs(d_ref[...])
    fout = jnp.dot(feat_ref[0], w_ref[...],
                   preferred_element_type=jnp.float32) + b_ref[0][None, :]
    center = fout[:, :out_c]
    support = fout[:, out_c:]
    acc = jnp.full((verts.shape[0], S * out_c), -INF, jnp.float32)
    vsplit = _split3(verts)
    for p in nbrs:
        ndn = _neighbor_dir_norm(verts, p, vsplit)
        theta = jax.nn.relu(jnp.dot(ndn, sdn,
                                    preferred_element_type=jnp.float32))
        fs = _gather_rows(support, p)
        acc = jnp.maximum(acc, theta * fs)
    y = center + _group_sum(acc, out_c)
    y_ref[0] = y
    if with_global:
        g_ref[0, 0] = jnp.max(y, axis=0)


def _argmin_cols(d):
    v, w = d.shape
    lane = jax.lax.broadcasted_iota(jnp.int32, (v, w), 1)
    rowmin = jnp.min(d, axis=1, keepdims=True)
    return jnp.min(jnp.where(d == rowmin, lane, w), axis=1, keepdims=True)


def _upsample_body(verts_ref, v1t_ref, v2t_ref, f2_ref, f3_ref, f4_ref,
                   o2_ref, o3_ref, o4_ref):
    verts = verts_ref[0]
    np1 = _argmin_cols(_cross_dist(verts, v1t_ref[0]))
    np2 = _argmin_cols(_cross_dist(verts, v2t_ref[0]))
    oh1 = _onehot(np1, v1t_ref.shape[2])
    oh2 = _onehot(np2, v2t_ref.shape[2])
    o2_ref[0] = jnp.dot(oh1, f2_ref[0], preferred_element_type=jnp.float32)
    o3_ref[0] = jnp.dot(oh1, f3_ref[0], preferred_element_type=jnp.float32)
    o4_ref[0] = jnp.dot(oh2, f4_ref[0], preferred_element_type=jnp.float32)


def _batched_spec(shape):
    return pl.BlockSpec((1,) + shape[1:], lambda b: (b,) + (0,) * (len(shape) - 1))


def _full_spec(shape):
    return pl.BlockSpec(shape, lambda b: (0,) * len(shape))


def _run_s1(vertices, d0, w1, b1, d1):
    bs, v, _ = vertices.shape
    vt = vertices.swapaxes(1, 2)
    return pl.pallas_call(
        _s1_body,
        grid=(bs,),
        in_specs=[_batched_spec(vertices.shape), _batched_spec(vt.shape),
                  _full_spec(d0.shape),
                  _full_spec(w1.shape), _full_spec((1, b1.shape[0])),
                  _full_spec(d1.shape)],
        out_specs=[_batched_spec((bs, v, 128)), _batched_spec((bs, v, 128)),
                   _batched_spec((bs, v, 8))],
        out_shape=[jax.ShapeDtypeStruct((bs, v, 128), jnp.float32),
                   jax.ShapeDtypeStruct((bs, v, 128), jnp.float32),
                   jax.ShapeDtypeStruct((bs, v, 8), jnp.int32)],
    )(vertices, vt, d0, w1, b1.reshape(1, -1), d1)


def _run_bn(y, g, be):
    return pl.pallas_call(
        _bn_body,
        grid=(1,),
        in_specs=[_full_spec(y.shape), _full_spec((1, g.shape[0])),
                  _full_spec((1, be.shape[0]))],
        out_specs=_full_spec(y.shape),
        out_shape=jax.ShapeDtypeStruct(y.shape, jnp.float32),
    )(y, g.reshape(1, -1), be.reshape(1, -1))


def _run_pool(fm, nidx4):
    bs, vp, _ = nidx4.shape
    c = fm.shape[2]
    return pl.pallas_call(
        _pool_body,
        grid=(bs,),
        in_specs=[_batched_spec(fm.shape), _batched_spec(nidx4.shape)],
        out_specs=_batched_spec((bs, vp, c)),
        out_shape=jax.ShapeDtypeStruct((bs, vp, c), jnp.float32),
    )(fm, nidx4)


def _run_conv(verts, feat, w, b, d, out_c, n_nbr, knn, with_global,
              nidx=None):
    bs, v, _ = verts.shape
    body = functools.partial(_conv_body, out_c=out_c, n_nbr=n_nbr, knn=knn,
                             with_global=with_global)
    vt = verts.swapaxes(1, 2)
    in_specs = [_batched_spec(verts.shape), _batched_spec(vt.shape),
                _batched_spec(feat.shape),
                _full_spec(w.shape), _full_spec((1, b.shape[0])),
                _full_spec(d.shape)]
    args = [verts, vt, feat, w, b.reshape(1, -1), d]
    out_specs = []
    out_shape = []
    if not knn:
        in_specs.append(_batched_spec(nidx.shape))
        args.append(nidx)
    out_specs.append(_batched_spec((bs, v, out_c)))
    out_shape.append(jax.ShapeDtypeStruct((bs, v, out_c), jnp.float32))
    if knn:
        out_specs.append(_batched_spec((bs, v, n_nbr)))
        out_shape.append(jax.ShapeDtypeStruct((bs, v, n_nbr), jnp.int32))
    if with_global:
        out_specs.append(pl.BlockSpec((1, 1, out_c), lambda bb: (bb, 0, 0)))
        out_shape.append(jax.ShapeDtypeStruct((bs, 1, out_c), jnp.float32))
    return pl.pallas_call(
        body, grid=(bs,), in_specs=in_specs, out_specs=out_specs,
        out_shape=out_shape,
    )(*args)


def _run_upsample(vertices, v1, v2, fm2, fm3, fm4):
    bs, v, _ = vertices.shape
    v1t = v1.swapaxes(1, 2)
    v2t = v2.swapaxes(1, 2)
    return pl.pallas_call(
        _upsample_body,
        grid=(bs,),
        in_specs=[_batched_spec(vertices.shape), _batched_spec(v1t.shape),
                  _batched_spec(v2t.shape), _batched_spec(fm2.shape),
                  _batched_spec(fm3.shape), _batched_spec(fm4.shape)],
        out_specs=[_batched_spec((bs, v, 256)), _batched_spec((bs, v, 256)),
                   _batched_spec((bs, v, 512))],
        out_shape=[jax.ShapeDtypeStruct((bs, v, 256), jnp.float32),
                   jax.ShapeDtypeStruct((bs, v, 256), jnp.float32),
                   jax.ShapeDtypeStruct((bs, v, 512), jnp.float32)],
    )(vertices, v1t, v2t, fm2, fm3, fm4)


def kernel(vertices, cat_id, d0, w1, b1, d1, g1, be1, w2, b2, d2, g2, be2,
           w3, b3, d3, g3, be3, w4, b4, d4):
    bs, vnum, _ = vertices.shape

    fm0, y1, nidx8 = _run_s1(vertices, d0, w1, b1, d1)
    fm1 = _run_bn(y1, g1, be1)

    sample1 = jax.random.permutation(jax.random.key(1), vnum)[:vnum // 4]
    v1 = vertices[:, sample1, :]
    fp1 = _run_pool(fm1, nidx8[:, sample1, :4])

    y2, nidx1 = _run_conv(v1, fp1, w2, b2, d2, out_c=256, n_nbr=10,
                          knn=True, with_global=False)
    fm2 = _run_bn(y2, g2, be2)

    y3 = _run_conv(v1, fm2, w3, b3, d3, out_c=256, n_nbr=10, knn=False,
                   with_global=False, nidx=nidx1)[0]
    fm3 = _run_bn(y3, g3, be3)

    sample2 = jax.random.permutation(jax.random.key(2), vnum // 4)[:vnum // 16]
    v2 = v1[:, sample2, :]
    fp2 = _run_pool(fm3, nidx1[:, sample2, :4])

    fm4, _, f_global = _run_conv(v2, fp2, w4, b4, d4, out_c=512, n_nbr=8,
                                 knn=True, with_global=True)
    f_global = f_global.reshape(bs, 512)

    fm2u, fm3u, fm4u = _run_upsample(vertices, v1, v2, fm2, fm3, fm4)

    one_hot = jnp.zeros((bs, OBJ_C), vertices.dtype).at[
        jnp.arange(bs), cat_id.reshape(-1)].set(1.0)
    oh = jnp.broadcast_to(one_hot[:, None, :], (bs, vnum, OBJ_C))
    feat = jnp.concatenate([fm0, fm1, fm2u, fm3u, fm4u, oh], axis=2)
    return feat, f_global
